# baseline (speedup 1.0000x reference)
import jax
import jax.numpy as jnp
from jax.experimental import pallas as pl
from jax.experimental.pallas import tpu as pltpu
from jax.scipy.linalg import block_diag

K = 3
R = 2
N = 512
KT = 3


def _dot(a, b):
    return jax.lax.dot_general(
        a, b, (((1,), (0,)), ((), ())),
        preferred_element_type=jnp.float32)


def _fused_kernel(x0_ref, x1_ref, x2_ref, sup_ref, att_ref, w1_ref,
                  wtc_ref, wlbd_ref, wrbd_ref, w2_ref, g_ref, bta_ref,
                  out_ref, mscr, hbuf):
    t = pl.program_id(1)
    first = jnp.logical_and(pl.program_id(0) == 0, t == 0)

    @pl.when(first)
    def _():
        m0 = (att_ref[0] != 0).astype(jnp.float32)
        m1 = (att_ref[1] != 0).astype(jnp.float32)
        mscr[0] = m0
        mscr[1] = 1.0 - m0
        mscr[2] = m1
        mscr[3] = 1.0 - m1
        for k in range(K):
            mk = (sup_ref[k] != 0).astype(jnp.float32)
            uk = ((m0 + m1 + mk) > 0).astype(jnp.float32)
            mscr[4 + k] = mk
            mscr[7 + k] = (1.0 - mk) * uk

    cin = x0_ref.shape[-1]
    w1 = w1_ref[...]
    y = _dot(x0_ref[0, 0], w1[:cin])
    y = y + _dot(x1_ref[0, 0], w1[cin:2 * cin])
    y = y + _dot(x2_ref[0, 0], w1[2 * cin:])
    ch = y.shape[-1] // 2
    h = y[:, :ch] * jax.nn.sigmoid(y[:, ch:])

    wxa = _dot(h, wtc_ref[...])
    eal = jnp.exp(_dot(wxa, wlbd_ref[...]))
    ear = jnp.exp(jax.lax.dot_general(
        wrbd_ref[...], wxa, (((0,), (1,)), ((), ())),
        preferred_element_type=jnp.float32))

    cs = wxa.shape[-1] // K
    ones = jnp.ones((N, 1), dtype=jnp.float32)
    m0 = mscr[0]
    nm0 = mscr[1]
    m1 = mscr[2]
    nm1 = mscr[3]
    attn = jnp.zeros((N, cs), dtype=jnp.float32)
    for k in range(K):
        mk = mscr[4 + k]
        wk = mscr[7 + k]
        c = (R + 1) * k
        f = (m0 * eal[:, c:c + 1]) * ear[c:c + 1, :] + nm0
        f = f * ((m1 * eal[:, c + 1:c + 2]) * ear[c + 1:c + 2, :] + nm1)
        f = f * ((mk * eal[:, c + 2:c + 3]) * ear[c + 2:c + 3, :] + wk)
        aug = jnp.concatenate([wxa[:, cs * k:cs * (k + 1)], ones], axis=1)
        ew = _dot(f, aug)
        attn = attn + (1.0 / ew[:, cs:cs + 1]) * ew[:, :cs]
    attn = jnp.where(attn > 0, attn, jnp.exp(jnp.minimum(attn, 0.0)) - 1.0)
    hbuf[t % 3] = attn

    @pl.when(t >= 2)
    def _():
        cs = hbuf.shape[-1]
        w2 = w2_ref[...]
        y2 = _dot(hbuf[(t + 1) % 3], w2[:cs])
        y2 = y2 + _dot(hbuf[(t + 2) % 3], w2[cs:2 * cs])
        y2 = y2 + _dot(hbuf[t % 3], w2[2 * cs:])
        co = y2.shape[-1] // 2
        g = y2[:, :co] * jax.nn.sigmoid(y2[:, co:])
        mu = jnp.mean(g)
        var = jnp.mean((g - mu) * (g - mu))
        out_ref[0, 0] = ((g - mu) / jnp.sqrt(var + 1e-6)) * g_ref[0, 0] \
            + bta_ref[0, 0]


def kernel(x, supports, atten_supports, W1, W_transform, W_left, W_right,
           W2, gamma, beta):
    B, T, n, cin = x.shape
    ch2 = W1.shape[-1]
    ch = ch2 // 2
    cs = W_transform.shape[-1]
    cout2 = W2.shape[-1]
    T1 = T - KT + 1
    T2 = T1 - KT + 1

    w1f = W1.reshape(KT * cin, ch2)
    wtc = jnp.moveaxis(W_transform, 0, 1).reshape(ch, K * cs)
    wlbd = block_diag(*[W_left[k].T for k in range(K)])
    wrbd = block_diag(*[W_right[k].T for k in range(K)])
    w2f = W2.reshape(KT * cs, cout2)

    out = pl.pallas_call(
        _fused_kernel,
        grid=(B, T1),
        in_specs=[
            pl.BlockSpec((1, 1, n, cin), lambda b, t: (b, t, 0, 0)),
            pl.BlockSpec((1, 1, n, cin), lambda b, t: (b, t + 1, 0, 0)),
            pl.BlockSpec((1, 1, n, cin), lambda b, t: (b, t + 2, 0, 0)),
            pl.BlockSpec((K, n, n), lambda b, t: (0, 0, 0)),
            pl.BlockSpec((R, n, n), lambda b, t: (0, 0, 0)),
            pl.BlockSpec((KT * cin, ch2), lambda b, t: (0, 0)),
            pl.BlockSpec((ch, K * cs), lambda b, t: (0, 0)),
            pl.BlockSpec((K * cs, K * (R + 1)), lambda b, t: (0, 0)),
            pl.BlockSpec((K * cs, K * (R + 1)), lambda b, t: (0, 0)),
            pl.BlockSpec((KT * cs, cout2), lambda b, t: (0, 0)),
            pl.BlockSpec((1, 1, n, cout2 // 2), lambda b, t: (0, 0, 0, 0)),
            pl.BlockSpec((1, 1, n, cout2 // 2), lambda b, t: (0, 0, 0, 0)),
        ],
        out_specs=pl.BlockSpec(
            (1, 1, n, cout2 // 2),
            lambda b, t: (b, jnp.maximum(t - 2, 0), 0, 0)),
        out_shape=jax.ShapeDtypeStruct((B, T2, n, cout2 // 2), jnp.float32),
        scratch_shapes=[
            pltpu.VMEM((7 + K, n, n), jnp.float32),
            pltpu.VMEM((3, n, cs), jnp.float32),
        ],
        compiler_params=pltpu.CompilerParams(
            dimension_semantics=("arbitrary", "arbitrary")),
    )(x, x, x, supports, atten_supports, w1f, wtc, wlbd, wrbd, w2f,
      gamma, beta)
    return out


# speedup vs baseline: 1.1847x; 1.1847x over previous
import jax
import jax.numpy as jnp
from jax.experimental import pallas as pl
from jax.experimental.pallas import tpu as pltpu
from jax.scipy.linalg import block_diag

K = 3
R = 2
N = 512
KT = 3


def _dot(a, b):
    return jax.lax.dot_general(
        a, b, (((1,), (0,)), ((), ())),
        preferred_element_type=jnp.float32)


def _fused_kernel(x0_ref, x1_ref, x2_ref, sup_ref, att_ref, w1_ref,
                  wtc_ref, wlbd_ref, wrbd_ref, w2_ref, g_ref, bta_ref,
                  out_ref, mscr, hbuf):
    t = pl.program_id(1)
    first = jnp.logical_and(pl.program_id(0) == 0, t == 0)

    @pl.when(first)
    def _():
        m0 = (att_ref[0] != 0).astype(jnp.float32)
        m1 = (att_ref[1] != 0).astype(jnp.float32)
        mscr[0] = m0.astype(jnp.bfloat16)
        mscr[1] = (1.0 - m0).astype(jnp.bfloat16)
        mscr[2] = m1.astype(jnp.bfloat16)
        mscr[3] = (1.0 - m1).astype(jnp.bfloat16)
        for k in range(K):
            mk = (sup_ref[k] != 0).astype(jnp.float32)
            uk = ((m0 + m1 + mk) > 0).astype(jnp.float32)
            mscr[4 + k] = mk.astype(jnp.bfloat16)
            mscr[7 + k] = ((1.0 - mk) * uk).astype(jnp.bfloat16)

    xc = jnp.concatenate([x0_ref[0, 0], x1_ref[0, 0], x2_ref[0, 0]], axis=-1)
    y = _dot(xc, w1_ref[...])
    ch = y.shape[-1] // 2
    h = y[:, :ch] * jax.nn.sigmoid(y[:, ch:])

    wxa = _dot(h, wtc_ref[...])
    eal = jnp.exp(_dot(wxa, wlbd_ref[...]))
    ear = jnp.exp(jax.lax.dot_general(
        wrbd_ref[...], wxa, (((0,), (1,)), ((), ())),
        preferred_element_type=jnp.float32))

    cs = wxa.shape[-1] // K
    ones = jnp.ones((N, 1), dtype=jnp.bfloat16)
    ealh = eal.astype(jnp.bfloat16)
    earh = ear.astype(jnp.bfloat16)
    wxah = wxa.astype(jnp.bfloat16)
    m0 = mscr[0]
    nm0 = mscr[1]
    m1 = mscr[2]
    nm1 = mscr[3]
    attn = jnp.zeros((N, cs), dtype=jnp.float32)
    for k in range(K):
        mk = mscr[4 + k]
        wk = mscr[7 + k]
        c = (R + 1) * k
        f = (m0 * ealh[:, c:c + 1]) * earh[c:c + 1, :] + nm0
        f = f * ((m1 * ealh[:, c + 1:c + 2]) * earh[c + 1:c + 2, :] + nm1)
        f = f * ((mk * ealh[:, c + 2:c + 3]) * earh[c + 2:c + 3, :] + wk)
        aug = jnp.concatenate([wxah[:, cs * k:cs * (k + 1)], ones], axis=1)
        ew = _dot(f, aug)
        attn = attn + (1.0 / ew[:, cs:cs + 1]) * ew[:, :cs]
    attn = jnp.where(attn > 0, attn, jnp.exp(jnp.minimum(attn, 0.0)) - 1.0)
    hbuf[t % 3] = attn

    @pl.when(t >= 2)
    def _():
        hc = jnp.concatenate(
            [hbuf[(t + 1) % 3], hbuf[(t + 2) % 3], hbuf[t % 3]], axis=-1)
        y2 = _dot(hc, w2_ref[...])
        co = y2.shape[-1] // 2
        g = y2[:, :co] * jax.nn.sigmoid(y2[:, co:])
        mu = jnp.mean(g)
        var = jnp.mean((g - mu) * (g - mu))
        out_ref[0, 0] = ((g - mu) / jnp.sqrt(var + 1e-6)) * g_ref[0, 0] \
            + bta_ref[0, 0]


def kernel(x, supports, atten_supports, W1, W_transform, W_left, W_right,
           W2, gamma, beta):
    B, T, n, cin = x.shape
    ch2 = W1.shape[-1]
    ch = ch2 // 2
    cs = W_transform.shape[-1]
    cout2 = W2.shape[-1]
    T1 = T - KT + 1
    T2 = T1 - KT + 1

    w1f = W1.reshape(KT * cin, ch2)
    wtc = jnp.moveaxis(W_transform, 0, 1).reshape(ch, K * cs)
    wlbd = block_diag(*[W_left[k].T for k in range(K)])
    wrbd = block_diag(*[W_right[k].T for k in range(K)])
    w2f = W2.reshape(KT * cs, cout2)

    out = pl.pallas_call(
        _fused_kernel,
        grid=(B, T1),
        in_specs=[
            pl.BlockSpec((1, 1, n, cin), lambda b, t: (b, t, 0, 0)),
            pl.BlockSpec((1, 1, n, cin), lambda b, t: (b, t + 1, 0, 0)),
            pl.BlockSpec((1, 1, n, cin), lambda b, t: (b, t + 2, 0, 0)),
            pl.BlockSpec((K, n, n), lambda b, t: (0, 0, 0)),
            pl.BlockSpec((R, n, n), lambda b, t: (0, 0, 0)),
            pl.BlockSpec((KT * cin, ch2), lambda b, t: (0, 0)),
            pl.BlockSpec((ch, K * cs), lambda b, t: (0, 0)),
            pl.BlockSpec((K * cs, K * (R + 1)), lambda b, t: (0, 0)),
            pl.BlockSpec((K * cs, K * (R + 1)), lambda b, t: (0, 0)),
            pl.BlockSpec((KT * cs, cout2), lambda b, t: (0, 0)),
            pl.BlockSpec((1, 1, n, cout2 // 2), lambda b, t: (0, 0, 0, 0)),
            pl.BlockSpec((1, 1, n, cout2 // 2), lambda b, t: (0, 0, 0, 0)),
        ],
        out_specs=pl.BlockSpec(
            (1, 1, n, cout2 // 2),
            lambda b, t: (b, jnp.maximum(t - 2, 0), 0, 0)),
        out_shape=jax.ShapeDtypeStruct((B, T2, n, cout2 // 2), jnp.float32),
        scratch_shapes=[
            pltpu.VMEM((7 + K, n, n), jnp.bfloat16),
            pltpu.VMEM((3, n, cs), jnp.float32),
        ],
        compiler_params=pltpu.CompilerParams(
            dimension_semantics=("arbitrary", "arbitrary")),
    )(x, x, x, supports, atten_supports, w1f, wtc, wlbd, wrbd, w2f,
      gamma, beta)
    return out


# speedup vs baseline: 1.1881x; 1.0028x over previous
import jax
import jax.numpy as jnp
from jax.experimental import pallas as pl
from jax.experimental.pallas import tpu as pltpu
from jax.scipy.linalg import block_diag

K = 3
R = 2
N = 512
KT = 3


def _dot(a, b):
    return jax.lax.dot_general(
        a, b, (((1,), (0,)), ((), ())),
        preferred_element_type=jnp.float32)


def _fused_kernel(x_ref, sup_ref, att_ref, w1_ref, wtc_ref, wlbd_ref,
                  wrbd_ref, w2_ref, g_ref, bta_ref, out_ref,
                  mscr, hscr, hbuf):
    t = pl.program_id(1)
    first = jnp.logical_and(pl.program_id(0) == 0, t == 0)

    @pl.when(first)
    def _():
        m0 = (att_ref[0] != 0).astype(jnp.float32)
        m1 = (att_ref[1] != 0).astype(jnp.float32)
        mscr[0] = m0.astype(jnp.bfloat16)
        mscr[1] = (1.0 - m0).astype(jnp.bfloat16)
        mscr[2] = m1.astype(jnp.bfloat16)
        mscr[3] = (1.0 - m1).astype(jnp.bfloat16)
        for k in range(K):
            mk = (sup_ref[k] != 0).astype(jnp.float32)
            uk = ((m0 + m1 + mk) > 0).astype(jnp.float32)
            mscr[4 + k] = mk.astype(jnp.bfloat16)
            mscr[7 + k] = ((1.0 - mk) * uk).astype(jnp.bfloat16)

    @pl.when(t == 0)
    def _():
        T = x_ref.shape[1]
        cin = x_ref.shape[-1]
        t1 = T - KT + 1
        w1 = w1_ref[...]
        xf = x_ref[0].reshape(T * N, cin)
        a0 = _dot(xf, w1[:cin])
        a1 = _dot(xf, w1[cin:2 * cin])
        a2 = _dot(xf, w1[2 * cin:])
        y = a0[:t1 * N] + a1[N:(t1 + 1) * N] + a2[2 * N:(t1 + 2) * N]
        ch = y.shape[-1] // 2
        h = y[:, :ch] * jax.nn.sigmoid(y[:, ch:])
        hscr[...] = h.reshape(t1, N, ch)

    h = hscr[t]
    ha = jnp.concatenate([h, jnp.ones((N, 1), dtype=jnp.float32)], axis=1)
    wxa = _dot(ha, wtc_ref[...])
    eal = jnp.exp(_dot(wxa, wlbd_ref[...]))
    ear = jnp.exp(jax.lax.dot_general(
        wrbd_ref[...], wxa, (((0,), (1,)), ((), ())),
        preferred_element_type=jnp.float32))

    cs1 = wxa.shape[-1] // K
    cs = cs1 - 1
    ealh = eal.astype(jnp.bfloat16)
    earh = ear.astype(jnp.bfloat16)
    wxah = wxa.astype(jnp.bfloat16)
    m0 = mscr[0]
    nm0 = mscr[1]
    m1 = mscr[2]
    nm1 = mscr[3]
    attn = jnp.zeros((N, cs), dtype=jnp.float32)
    for k in range(K):
        mk = mscr[4 + k]
        wk = mscr[7 + k]
        c = (R + 1) * k
        f = (m0 * ealh[:, c:c + 1]) * earh[c:c + 1, :] + nm0
        f = f * ((m1 * ealh[:, c + 1:c + 2]) * earh[c + 1:c + 2, :] + nm1)
        f = f * ((mk * ealh[:, c + 2:c + 3]) * earh[c + 2:c + 3, :] + wk)
        ew = _dot(f, wxah[:, cs1 * k:cs1 * (k + 1)])
        attn = attn + (1.0 / ew[:, cs:cs + 1]) * ew[:, :cs]
    attn = jnp.where(attn > 0, attn, jnp.exp(jnp.minimum(attn, 0.0)) - 1.0)
    hbuf[t % 3] = attn

    @pl.when(t >= 2)
    def _():
        hc = jnp.concatenate(
            [hbuf[(t + 1) % 3], hbuf[(t + 2) % 3], hbuf[t % 3]], axis=-1)
        y2 = _dot(hc, w2_ref[...])
        co = y2.shape[-1] // 2
        g = y2[:, :co] * jax.nn.sigmoid(y2[:, co:])
        mu = jnp.mean(g)
        var = jnp.mean((g - mu) * (g - mu))
        out_ref[0, 0] = ((g - mu) / jnp.sqrt(var + 1e-6)) * g_ref[0, 0] \
            + bta_ref[0, 0]


def kernel(x, supports, atten_supports, W1, W_transform, W_left, W_right,
           W2, gamma, beta):
    B, T, n, cin = x.shape
    ch2 = W1.shape[-1]
    ch = ch2 // 2
    cs = W_transform.shape[-1]
    cout2 = W2.shape[-1]
    T1 = T - KT + 1
    T2 = T1 - KT + 1

    w1f = W1.reshape(KT * cin, ch2)
    onecol = jnp.zeros((ch + 1, 1), jnp.float32).at[ch, 0].set(1.0)
    blocks = []
    for k in range(K):
        wk = jnp.concatenate(
            [W_transform[k], jnp.zeros((1, cs), jnp.float32)], axis=0)
        blocks.append(jnp.concatenate([wk, onecol], axis=1))
    wtc_aug = jnp.concatenate(blocks, axis=1)
    zrow = jnp.zeros((1, R + 1), jnp.float32)
    wlbd = block_diag(*[jnp.concatenate([W_left[k].T, zrow], axis=0)
                        for k in range(K)])
    wrbd = block_diag(*[jnp.concatenate([W_right[k].T, zrow], axis=0)
                        for k in range(K)])
    w2f = W2.reshape(KT * cs, cout2)

    out = pl.pallas_call(
        _fused_kernel,
        grid=(B, T1),
        in_specs=[
            pl.BlockSpec((1, T, n, cin), lambda b, t: (b, 0, 0, 0)),
            pl.BlockSpec((K, n, n), lambda b, t: (0, 0, 0)),
            pl.BlockSpec((R, n, n), lambda b, t: (0, 0, 0)),
            pl.BlockSpec((KT * cin, ch2), lambda b, t: (0, 0)),
            pl.BlockSpec((ch + 1, K * (cs + 1)), lambda b, t: (0, 0)),
            pl.BlockSpec((K * (cs + 1), K * (R + 1)), lambda b, t: (0, 0)),
            pl.BlockSpec((K * (cs + 1), K * (R + 1)), lambda b, t: (0, 0)),
            pl.BlockSpec((KT * cs, cout2), lambda b, t: (0, 0)),
            pl.BlockSpec((1, 1, n, cout2 // 2), lambda b, t: (0, 0, 0, 0)),
            pl.BlockSpec((1, 1, n, cout2 // 2), lambda b, t: (0, 0, 0, 0)),
        ],
        out_specs=pl.BlockSpec(
            (1, 1, n, cout2 // 2),
            lambda b, t: (b, jnp.maximum(t - 2, 0), 0, 0)),
        out_shape=jax.ShapeDtypeStruct((B, T2, n, cout2 // 2), jnp.float32),
        scratch_shapes=[
            pltpu.VMEM((7 + K, n, n), jnp.bfloat16),
            pltpu.VMEM((T1, n, ch), jnp.float32),
            pltpu.VMEM((3, n, cs), jnp.float32),
        ],
        compiler_params=pltpu.CompilerParams(
            dimension_semantics=("arbitrary", "arbitrary")),
    )(x, supports, atten_supports, w1f, wtc_aug, wlbd, wrbd, w2f,
      gamma, beta)
    return out


# speedup vs baseline: 1.2421x; 1.0454x over previous
import jax
import jax.numpy as jnp
from jax.experimental import pallas as pl
from jax.experimental.pallas import tpu as pltpu

K = 3
R = 2
N = 512
KT = 3


def _dot(a, b):
    return jax.lax.dot_general(
        a, b, (((1,), (0,)), ((), ())),
        preferred_element_type=jnp.float32)


def _fused_kernel(x_ref, sup_ref, att_ref, w1_ref, wt_ref, wlr_ref,
                  w2_ref, g_ref, bta_ref, out_ref,
                  mscr, hscr, hbuf, wtc_scr, wl_scr, wr_scr):
    t = pl.program_id(1)
    first = jnp.logical_and(pl.program_id(0) == 0, t == 0)

    ch = wt_ref.shape[1]
    cs = wt_ref.shape[2]
    cs1 = cs + 1
    r1 = R + 1

    @pl.when(first)
    def _():
        m0 = (att_ref[0] != 0).astype(jnp.float32)
        m1 = (att_ref[1] != 0).astype(jnp.float32)
        mscr[0] = m0.astype(jnp.bfloat16)
        mscr[1] = (1.0 - m0).astype(jnp.bfloat16)
        mscr[2] = m1.astype(jnp.bfloat16)
        mscr[3] = (1.0 - m1).astype(jnp.bfloat16)
        for k in range(K):
            mk = (sup_ref[k] != 0).astype(jnp.float32)
            uk = ((m0 + m1 + mk) > 0).astype(jnp.float32)
            mscr[4 + k] = mk.astype(jnp.bfloat16)
            mscr[7 + k] = ((1.0 - mk) * uk).astype(jnp.bfloat16)
        col = jax.lax.broadcasted_iota(jnp.int32, (ch + 1, K * cs1), 1)
        row = jax.lax.broadcasted_iota(jnp.int32, (ch + 1, K * cs1), 0)
        wtc_scr[...] = jnp.where(
            jnp.logical_and((col % cs1) == cs, row == ch), 1.0, 0.0)
        wl_scr[...] = jnp.zeros((K * cs1, K * r1), jnp.float32)
        wr_scr[...] = jnp.zeros((K * cs1, K * r1), jnp.float32)
        for k in range(K):
            wtc_scr[0:ch, cs1 * k:cs1 * k + cs] = wt_ref[k]
            wl_scr[cs1 * k:cs1 * k + cs, r1 * k:r1 * (k + 1)] = wlr_ref[0, k]
            wr_scr[cs1 * k:cs1 * k + cs, r1 * k:r1 * (k + 1)] = wlr_ref[1, k]

    @pl.when(t == 0)
    def _():
        T = x_ref.shape[1]
        cin = x_ref.shape[-1]
        t1 = T - KT + 1
        xf = x_ref[0].reshape(T * N, cin)
        a0 = _dot(xf, w1_ref[0])
        a1 = _dot(xf, w1_ref[1])
        a2 = _dot(xf, w1_ref[2])
        y = a0[:t1 * N] + a1[N:(t1 + 1) * N] + a2[2 * N:(t1 + 2) * N]
        c2 = y.shape[-1] // 2
        h = y[:, :c2] * jax.nn.sigmoid(y[:, c2:])
        hscr[...] = jnp.concatenate(
            [h, jnp.ones((t1 * N, 1), jnp.float32)],
            axis=1).reshape(t1, N, c2 + 1)

    ha = hscr[t]
    wxa = _dot(ha, wtc_scr[...])
    eal = jnp.exp(_dot(wxa, wl_scr[...]))
    ear = jnp.exp(jax.lax.dot_general(
        wr_scr[...], wxa, (((0,), (1,)), ((), ())),
        preferred_element_type=jnp.float32))

    ealh = eal.astype(jnp.bfloat16)
    earh = ear.astype(jnp.bfloat16)
    wxah = wxa.astype(jnp.bfloat16)
    m0 = mscr[0]
    nm0 = mscr[1]
    m1 = mscr[2]
    nm1 = mscr[3]
    attn = jnp.zeros((N, cs), dtype=jnp.float32)
    for k in range(K):
        mk = mscr[4 + k]
        wk = mscr[7 + k]
        c = r1 * k
        f = (m0 * ealh[:, c:c + 1]) * earh[c:c + 1, :] + nm0
        f = f * ((m1 * ealh[:, c + 1:c + 2]) * earh[c + 1:c + 2, :] + nm1)
        f = f * ((mk * ealh[:, c + 2:c + 3]) * earh[c + 2:c + 3, :] + wk)
        ew = _dot(f, wxah[:, cs1 * k:cs1 * (k + 1)])
        attn = attn + (1.0 / ew[:, cs:cs + 1]) * ew[:, :cs]
    attn = jnp.where(attn > 0, attn, jnp.exp(jnp.minimum(attn, 0.0)) - 1.0)
    hbuf[t % 3] = attn

    @pl.when(t >= 2)
    def _():
        y2 = _dot(hbuf[(t + 1) % 3], w2_ref[0])
        y2 = y2 + _dot(hbuf[(t + 2) % 3], w2_ref[1])
        y2 = y2 + _dot(hbuf[t % 3], w2_ref[2])
        co = y2.shape[-1] // 2
        g = y2[:, :co] * jax.nn.sigmoid(y2[:, co:])
        mu = jnp.mean(g)
        var = jnp.mean((g - mu) * (g - mu))
        out_ref[0, 0] = ((g - mu) / jnp.sqrt(var + 1e-6)) * g_ref[0, 0] \
            + bta_ref[0, 0]


def kernel(x, supports, atten_supports, W1, W_transform, W_left, W_right,
           W2, gamma, beta):
    B, T, n, cin = x.shape
    ch2 = W1.shape[-1]
    ch = ch2 // 2
    cs = W_transform.shape[-1]
    cout2 = W2.shape[-1]
    T1 = T - KT + 1
    T2 = T1 - KT + 1

    wlr = jnp.stack([jnp.moveaxis(W_left, 1, 2),
                     jnp.moveaxis(W_right, 1, 2)])

    out = pl.pallas_call(
        _fused_kernel,
        grid=(B, T1),
        in_specs=[
            pl.BlockSpec((1, T, n, cin), lambda b, t: (b, 0, 0, 0)),
            pl.BlockSpec((K, n, n), lambda b, t: (0, 0, 0)),
            pl.BlockSpec((R, n, n), lambda b, t: (0, 0, 0)),
            pl.BlockSpec((KT, cin, ch2), lambda b, t: (0, 0, 0)),
            pl.BlockSpec((K, ch, cs), lambda b, t: (0, 0, 0)),
            pl.BlockSpec((2, K, cs, R + 1), lambda b, t: (0, 0, 0, 0)),
            pl.BlockSpec((KT, cs, cout2), lambda b, t: (0, 0, 0)),
            pl.BlockSpec((1, 1, n, cout2 // 2), lambda b, t: (0, 0, 0, 0)),
            pl.BlockSpec((1, 1, n, cout2 // 2), lambda b, t: (0, 0, 0, 0)),
        ],
        out_specs=pl.BlockSpec(
            (1, 1, n, cout2 // 2),
            lambda b, t: (b, jnp.maximum(t - 2, 0), 0, 0)),
        out_shape=jax.ShapeDtypeStruct((B, T2, n, cout2 // 2), jnp.float32),
        scratch_shapes=[
            pltpu.VMEM((7 + K, n, n), jnp.bfloat16),
            pltpu.VMEM((T1, n, ch + 1), jnp.float32),
            pltpu.VMEM((3, n, cs), jnp.float32),
            pltpu.VMEM((ch + 1, K * (cs + 1)), jnp.float32),
            pltpu.VMEM((K * (cs + 1), K * (R + 1)), jnp.float32),
            pltpu.VMEM((K * (cs + 1), K * (R + 1)), jnp.float32),
        ],
        compiler_params=pltpu.CompilerParams(
            dimension_semantics=("arbitrary", "arbitrary")),
    )(x, supports, atten_supports, W1, W_transform, wlr, W2, gamma, beta)
    return out


# speedup vs baseline: 1.2444x; 1.0019x over previous
import jax
import jax.numpy as jnp
from jax.experimental import pallas as pl
from jax.experimental.pallas import tpu as pltpu

K = 3
R = 2
N = 512
KT = 3


def _dot(a, b):
    return jax.lax.dot_general(
        a, b, (((1,), (0,)), ((), ())),
        preferred_element_type=jnp.float32)


def _fused_kernel(x_ref, sup_ref, att_ref, w1_ref, wt_ref, wlf_ref, wrf_ref,
                  w2_ref, g_ref, bta_ref, out_ref,
                  mscr, hscr, hbuf, wtc_scr, wl_scr, wr_scr):
    t = pl.program_id(1)
    first = jnp.logical_and(pl.program_id(0) == 0, t == 0)

    ch = wt_ref.shape[1]
    cs = wt_ref.shape[2]
    cs1 = cs + 1
    r1 = R + 1

    @pl.when(first)
    def _():
        m0 = (att_ref[0] != 0).astype(jnp.float32)
        m1 = (att_ref[1] != 0).astype(jnp.float32)
        mscr[0] = m0.astype(jnp.bfloat16)
        mscr[1] = (1.0 - m0).astype(jnp.bfloat16)
        mscr[2] = m1.astype(jnp.bfloat16)
        mscr[3] = (1.0 - m1).astype(jnp.bfloat16)
        for k in range(K):
            mk = (sup_ref[k] != 0).astype(jnp.float32)
            uk = ((m0 + m1 + mk) > 0).astype(jnp.float32)
            mscr[4 + k] = mk.astype(jnp.bfloat16)
            mscr[7 + k] = ((1.0 - mk) * uk).astype(jnp.bfloat16)
        col = jax.lax.broadcasted_iota(jnp.int32, (ch + 1, K * cs1), 1)
        row = jax.lax.broadcasted_iota(jnp.int32, (ch + 1, K * cs1), 0)
        wtc_scr[...] = jnp.where(
            jnp.logical_and((col % cs1) == cs, row == ch), 1.0, 0.0)
        wl_scr[...] = jnp.zeros((K * r1, K * cs1), jnp.float32)
        wr_scr[...] = jnp.zeros((K * r1, K * cs1), jnp.float32)
        for k in range(K):
            wtc_scr[0:ch, cs1 * k:cs1 * k + cs] = wt_ref[k]
            wl_scr[r1 * k:r1 * (k + 1), cs1 * k:cs1 * k + cs] = wlf_ref[k]
            wr_scr[r1 * k:r1 * (k + 1), cs1 * k:cs1 * k + cs] = wrf_ref[k]

    @pl.when(t == 0)
    def _():
        T = x_ref.shape[1]
        cin = x_ref.shape[-1]
        t1 = T - KT + 1
        xf = x_ref[0].reshape(T * N, cin)
        a0 = _dot(xf, w1_ref[0])
        a1 = _dot(xf, w1_ref[1])
        a2 = _dot(xf, w1_ref[2])
        y = a0[:t1 * N] + a1[N:(t1 + 1) * N] + a2[2 * N:(t1 + 2) * N]
        c2 = y.shape[-1] // 2
        h = y[:, :c2] * jax.nn.sigmoid(y[:, c2:])
        hscr[...] = jnp.concatenate(
            [h, jnp.ones((t1 * N, 1), jnp.float32)],
            axis=1).reshape(t1, N, c2 + 1)

    ha = hscr[t]
    wxa = _dot(ha, wtc_scr[...])
    eal = jnp.exp(jax.lax.dot_general(
        wxa, wl_scr[...], (((1,), (1,)), ((), ())),
        preferred_element_type=jnp.float32))
    ear = jnp.exp(jax.lax.dot_general(
        wr_scr[...], wxa, (((1,), (1,)), ((), ())),
        preferred_element_type=jnp.float32))

    ealh = eal.astype(jnp.bfloat16)
    earh = ear.astype(jnp.bfloat16)
    wxah = wxa.astype(jnp.bfloat16)
    m0 = mscr[0]
    nm0 = mscr[1]
    m1 = mscr[2]
    nm1 = mscr[3]
    attn = jnp.zeros((N, cs), dtype=jnp.float32)
    for k in range(K):
        mk = mscr[4 + k]
        wk = mscr[7 + k]
        c = r1 * k
        f = (m0 * ealh[:, c:c + 1]) * earh[c:c + 1, :] + nm0
        f = f * ((m1 * ealh[:, c + 1:c + 2]) * earh[c + 1:c + 2, :] + nm1)
        f = f * ((mk * ealh[:, c + 2:c + 3]) * earh[c + 2:c + 3, :] + wk)
        ew = _dot(f, wxah[:, cs1 * k:cs1 * (k + 1)])
        attn = attn + (1.0 / ew[:, cs:cs + 1]) * ew[:, :cs]
    attn = jnp.where(attn > 0, attn, jnp.exp(jnp.minimum(attn, 0.0)) - 1.0)
    hbuf[t % 3] = attn

    @pl.when(t >= 2)
    def _():
        y2 = _dot(hbuf[(t + 1) % 3], w2_ref[0])
        y2 = y2 + _dot(hbuf[(t + 2) % 3], w2_ref[1])
        y2 = y2 + _dot(hbuf[t % 3], w2_ref[2])
        co = y2.shape[-1] // 2
        g = y2[:, :co] * jax.nn.sigmoid(y2[:, co:])
        mu = jnp.mean(g)
        var = jnp.mean((g - mu) * (g - mu))
        out_ref[0, 0] = ((g - mu) / jnp.sqrt(var + 1e-6)) * g_ref[0, 0] \
            + bta_ref[0, 0]


def kernel(x, supports, atten_supports, W1, W_transform, W_left, W_right,
           W2, gamma, beta):
    B, T, n, cin = x.shape
    ch2 = W1.shape[-1]
    ch = ch2 // 2
    cs = W_transform.shape[-1]
    cout2 = W2.shape[-1]
    T1 = T - KT + 1
    T2 = T1 - KT + 1

    out = pl.pallas_call(
        _fused_kernel,
        grid=(B, T1),
        in_specs=[
            pl.BlockSpec((1, T, n, cin), lambda b, t: (b, 0, 0, 0)),
            pl.BlockSpec((K, n, n), lambda b, t: (0, 0, 0)),
            pl.BlockSpec((R, n, n), lambda b, t: (0, 0, 0)),
            pl.BlockSpec((KT, cin, ch2), lambda b, t: (0, 0, 0)),
            pl.BlockSpec((K, ch, cs), lambda b, t: (0, 0, 0)),
            pl.BlockSpec((K, R + 1, cs), lambda b, t: (0, 0, 0)),
            pl.BlockSpec((K, R + 1, cs), lambda b, t: (0, 0, 0)),
            pl.BlockSpec((KT, cs, cout2), lambda b, t: (0, 0, 0)),
            pl.BlockSpec((1, 1, n, cout2 // 2), lambda b, t: (0, 0, 0, 0)),
            pl.BlockSpec((1, 1, n, cout2 // 2), lambda b, t: (0, 0, 0, 0)),
        ],
        out_specs=pl.BlockSpec(
            (1, 1, n, cout2 // 2),
            lambda b, t: (b, jnp.maximum(t - 2, 0), 0, 0)),
        out_shape=jax.ShapeDtypeStruct((B, T2, n, cout2 // 2), jnp.float32),
        scratch_shapes=[
            pltpu.VMEM((7 + K, n, n), jnp.bfloat16),
            pltpu.VMEM((T1, n, ch + 1), jnp.float32),
            pltpu.VMEM((3, n, cs), jnp.float32),
            pltpu.VMEM((ch + 1, K * (cs + 1)), jnp.float32),
            pltpu.VMEM((K * (R + 1), K * (cs + 1)), jnp.float32),
            pltpu.VMEM((K * (R + 1), K * (cs + 1)), jnp.float32),
        ],
        compiler_params=pltpu.CompilerParams(
            dimension_semantics=("arbitrary", "arbitrary")),
    )(x, supports, atten_supports, W1, W_transform, W_left, W_right, W2,
      gamma, beta)
    return out


# speedup vs baseline: 1.3844x; 1.1125x over previous
import jax
import jax.numpy as jnp
from jax.experimental import pallas as pl
from jax.experimental.pallas import tpu as pltpu

K = 3
R = 2
N = 512
KT = 3


def _dot(a, b):
    return jax.lax.dot_general(
        a, b, (((1,), (0,)), ((), ())),
        preferred_element_type=jnp.float32)


def _fused_kernel(x_ref, sup_ref, att_ref, w1_ref, wt_ref, wlf_ref, wrf_ref,
                  w2_ref, g_ref, bta_ref, out_ref,
                  mscr, hscr, hbuf, wtc_scr, wl_scr, wr_scr):
    j = pl.program_id(1)
    first = jnp.logical_and(pl.program_id(0) == 0, j == 0)

    ch = wt_ref.shape[1]
    cs = wt_ref.shape[2]
    cs1 = cs + 1
    r1 = R + 1

    @pl.when(first)
    def _():
        m0 = (att_ref[0] != 0).astype(jnp.float32)
        m1 = (att_ref[1] != 0).astype(jnp.float32)
        mscr[0] = m0.astype(jnp.bfloat16)
        mscr[1] = (1.0 - m0).astype(jnp.bfloat16)
        mscr[2] = m1.astype(jnp.bfloat16)
        mscr[3] = (1.0 - m1).astype(jnp.bfloat16)
        for k in range(K):
            mk = (sup_ref[k] != 0).astype(jnp.float32)
            uk = ((m0 + m1 + mk) > 0).astype(jnp.float32)
            mscr[4 + k] = mk.astype(jnp.bfloat16)
            mscr[7 + k] = ((1.0 - mk) * uk).astype(jnp.bfloat16)
        col = jax.lax.broadcasted_iota(jnp.int32, (ch + 1, K * cs1), 1)
        row = jax.lax.broadcasted_iota(jnp.int32, (ch + 1, K * cs1), 0)
        wtc_scr[...] = jnp.where(
            jnp.logical_and((col % cs1) == cs, row == ch), 1.0, 0.0)
        wl_scr[...] = jnp.zeros((K * r1, K * cs1), jnp.float32)
        wr_scr[...] = jnp.zeros((K * r1, K * cs1), jnp.float32)
        for k in range(K):
            wtc_scr[0:ch, cs1 * k:cs1 * k + cs] = wt_ref[k]
            wl_scr[r1 * k:r1 * (k + 1), cs1 * k:cs1 * k + cs] = wlf_ref[k]
            wr_scr[r1 * k:r1 * (k + 1), cs1 * k:cs1 * k + cs] = wrf_ref[k]

    @pl.when(j == 0)
    def _():
        T = x_ref.shape[1]
        cin = x_ref.shape[-1]
        t1 = T - KT + 1
        xf = x_ref[0].reshape(T * N, cin)
        a0 = _dot(xf, w1_ref[0])
        a1 = _dot(xf, w1_ref[1])
        a2 = _dot(xf, w1_ref[2])
        y = a0[:t1 * N] + a1[N:(t1 + 1) * N] + a2[2 * N:(t1 + 2) * N]
        c2 = y.shape[-1] // 2
        h = y[:, :c2] * jax.nn.sigmoid(y[:, c2:])
        hscr[...] = jnp.concatenate(
            [h, jnp.ones((t1 * N, 1), jnp.float32)],
            axis=1).reshape(t1, N, c2 + 1)

    m0 = mscr[0]
    nm0 = mscr[1]
    m1 = mscr[2]
    nm1 = mscr[3]

    def _attn_step(tt):
        ha = hscr[tt]
        wxa = _dot(ha, wtc_scr[...])
        eal = jnp.exp(jax.lax.dot_general(
            wxa, wl_scr[...], (((1,), (1,)), ((), ())),
            preferred_element_type=jnp.float32))
        ear = jnp.exp(jax.lax.dot_general(
            wr_scr[...], wxa, (((1,), (1,)), ((), ())),
            preferred_element_type=jnp.float32))
        ealh = eal.astype(jnp.bfloat16)
        earh = ear.astype(jnp.bfloat16)
        wxah = wxa.astype(jnp.bfloat16)
        attn = jnp.zeros((N, cs), dtype=jnp.float32)
        for k in range(K):
            mk = mscr[4 + k]
            wk = mscr[7 + k]
            c = r1 * k
            f = (m0 * ealh[:, c:c + 1]) * earh[c:c + 1, :] + nm0
            f = f * ((m1 * ealh[:, c + 1:c + 2]) * earh[c + 1:c + 2, :] + nm1)
            f = f * ((mk * ealh[:, c + 2:c + 3]) * earh[c + 2:c + 3, :] + wk)
            ew = _dot(f, wxah[:, cs1 * k:cs1 * (k + 1)])
            attn = attn + (1.0 / ew[:, cs:cs + 1]) * ew[:, :cs]
        attn = jnp.where(attn > 0, attn,
                         jnp.exp(jnp.minimum(attn, 0.0)) - 1.0)
        hbuf[tt % 4] = attn

    def _conv2_out(tt, slot):
        y2 = _dot(hbuf[tt % 4], w2_ref[0])
        y2 = y2 + _dot(hbuf[(tt + 1) % 4], w2_ref[1])
        y2 = y2 + _dot(hbuf[(tt + 2) % 4], w2_ref[2])
        co = y2.shape[-1] // 2
        g = y2[:, :co] * jax.nn.sigmoid(y2[:, co:])
        mu = jnp.mean(g)
        var = jnp.mean((g - mu) * (g - mu))
        out_ref[0, slot] = ((g - mu) / jnp.sqrt(var + 1e-6)) * g_ref[0, 0] \
            + bta_ref[0, 0]

    _attn_step(2 * j)
    _attn_step(2 * j + 1)

    @pl.when(j >= 1)
    def _():
        _conv2_out(2 * j - 2, 0)
        _conv2_out(2 * j - 1, 1)


def kernel(x, supports, atten_supports, W1, W_transform, W_left, W_right,
           W2, gamma, beta):
    B, T, n, cin = x.shape
    ch2 = W1.shape[-1]
    ch = ch2 // 2
    cs = W_transform.shape[-1]
    cout2 = W2.shape[-1]
    T1 = T - KT + 1
    T2 = T1 - KT + 1

    out = pl.pallas_call(
        _fused_kernel,
        grid=(B, T1 // 2),
        in_specs=[
            pl.BlockSpec((1, T, n, cin), lambda b, t: (b, 0, 0, 0)),
            pl.BlockSpec((K, n, n), lambda b, t: (0, 0, 0)),
            pl.BlockSpec((R, n, n), lambda b, t: (0, 0, 0)),
            pl.BlockSpec((KT, cin, ch2), lambda b, t: (0, 0, 0)),
            pl.BlockSpec((K, ch, cs), lambda b, t: (0, 0, 0)),
            pl.BlockSpec((K, R + 1, cs), lambda b, t: (0, 0, 0)),
            pl.BlockSpec((K, R + 1, cs), lambda b, t: (0, 0, 0)),
            pl.BlockSpec((KT, cs, cout2), lambda b, t: (0, 0, 0)),
            pl.BlockSpec((1, 1, n, cout2 // 2), lambda b, t: (0, 0, 0, 0)),
            pl.BlockSpec((1, 1, n, cout2 // 2), lambda b, t: (0, 0, 0, 0)),
        ],
        out_specs=pl.BlockSpec(
            (1, 2, n, cout2 // 2),
            lambda b, j: (b, jnp.maximum(j - 1, 0), 0, 0)),
        out_shape=jax.ShapeDtypeStruct((B, T2, n, cout2 // 2), jnp.float32),
        scratch_shapes=[
            pltpu.VMEM((7 + K, n, n), jnp.bfloat16),
            pltpu.VMEM((T1, n, ch + 1), jnp.float32),
            pltpu.VMEM((4, n, cs), jnp.float32),
            pltpu.VMEM((ch + 1, K * (cs + 1)), jnp.float32),
            pltpu.VMEM((K * (R + 1), K * (cs + 1)), jnp.float32),
            pltpu.VMEM((K * (R + 1), K * (cs + 1)), jnp.float32),
        ],
        compiler_params=pltpu.CompilerParams(
            dimension_semantics=("arbitrary", "arbitrary")),
    )(x, supports, atten_supports, W1, W_transform, W_left, W_right, W2,
      gamma, beta)
    return out


# speedup vs baseline: 1.5805x; 1.1417x over previous
import jax
import jax.numpy as jnp
from jax.experimental import pallas as pl
from jax.experimental.pallas import tpu as pltpu

K = 3
R = 2
N = 512
KT = 3


def _dot(a, b):
    return jax.lax.dot_general(
        a, b, (((1,), (0,)), ((), ())),
        preferred_element_type=jnp.float32)


def _fused_kernel(x_ref, sup_ref, att_ref, w1_ref, wt_ref, wlf_ref, wrf_ref,
                  w2_ref, g_ref, bta_ref, out_ref,
                  mscr, hscr, hbuf, wtc_scr, wl_scr, wr_scr):
    j = pl.program_id(1)
    first = jnp.logical_and(pl.program_id(0) == 0, j == 0)

    ch = wt_ref.shape[1]
    cs = wt_ref.shape[2]
    cs1 = cs + 1
    r1 = R + 1

    @pl.when(first)
    def _():
        m0 = (att_ref[0] != 0).astype(jnp.float32)
        m1 = (att_ref[1] != 0).astype(jnp.float32)
        mscr[0] = m0.astype(jnp.bfloat16)
        mscr[1] = (1.0 - m0).astype(jnp.bfloat16)
        mscr[2] = m1.astype(jnp.bfloat16)
        mscr[3] = (1.0 - m1).astype(jnp.bfloat16)
        for k in range(K):
            mk = (sup_ref[k] != 0).astype(jnp.float32)
            uk = ((m0 + m1 + mk) > 0).astype(jnp.float32)
            mscr[4 + k] = mk.astype(jnp.bfloat16)
            mscr[7 + k] = ((1.0 - mk) * uk).astype(jnp.bfloat16)
        col = jax.lax.broadcasted_iota(jnp.int32, (ch + 1, K * cs1), 1)
        row = jax.lax.broadcasted_iota(jnp.int32, (ch + 1, K * cs1), 0)
        wtc_scr[...] = jnp.where(
            jnp.logical_and((col % cs1) == cs, row == ch), 1.0, 0.0)
        wl_scr[...] = jnp.zeros((K * r1, K * cs1), jnp.float32)
        wr_scr[...] = jnp.zeros((K * r1, K * cs1), jnp.float32)
        for k in range(K):
            wtc_scr[0:ch, cs1 * k:cs1 * k + cs] = wt_ref[k]
            wl_scr[r1 * k:r1 * (k + 1), cs1 * k:cs1 * k + cs] = wlf_ref[k]
            wr_scr[r1 * k:r1 * (k + 1), cs1 * k:cs1 * k + cs] = wrf_ref[k]

    @pl.when(j == 0)
    def _():
        T = x_ref.shape[1]
        cin = x_ref.shape[-1]
        t1 = T - KT + 1
        xf = x_ref[0].reshape(T * N, cin)
        a0 = _dot(xf, w1_ref[0])
        a1 = _dot(xf, w1_ref[1])
        a2 = _dot(xf, w1_ref[2])
        y = a0[:t1 * N] + a1[N:(t1 + 1) * N] + a2[2 * N:(t1 + 2) * N]
        c2 = y.shape[-1] // 2
        h = y[:, :c2] * jax.nn.sigmoid(y[:, c2:])
        hscr[...] = jnp.concatenate(
            [h, jnp.ones((t1 * N, 1), jnp.float32)],
            axis=1).reshape(t1, N, c2 + 1)

    m0 = mscr[0]
    nm0 = mscr[1]
    m1 = mscr[2]
    nm1 = mscr[3]

    def _attn_step(tt):
        ha = hscr[tt]
        wxa = _dot(ha, wtc_scr[...])
        eal = jnp.exp(jax.lax.dot_general(
            wxa, wl_scr[...], (((1,), (1,)), ((), ())),
            preferred_element_type=jnp.float32))
        ear = jnp.exp(jax.lax.dot_general(
            wr_scr[...], wxa, (((1,), (1,)), ((), ())),
            preferred_element_type=jnp.float32))
        ealh = eal.astype(jnp.bfloat16)
        earh = ear.astype(jnp.bfloat16)
        wxah = wxa.astype(jnp.bfloat16)
        attn = jnp.zeros((N, cs), dtype=jnp.float32)
        for k in range(K):
            mk = mscr[4 + k]
            wk = mscr[7 + k]
            c = r1 * k
            f = (m0 * ealh[:, c:c + 1]) * earh[c:c + 1, :] + nm0
            f = f * ((m1 * ealh[:, c + 1:c + 2]) * earh[c + 1:c + 2, :] + nm1)
            f = f * ((mk * ealh[:, c + 2:c + 3]) * earh[c + 2:c + 3, :] + wk)
            ew = _dot(f, wxah[:, cs1 * k:cs1 * (k + 1)])
            attn = attn + (1.0 / ew[:, cs:cs + 1]) * ew[:, :cs]
        attn = jnp.where(attn > 0, attn,
                         jnp.exp(jnp.minimum(attn, 0.0)) - 1.0)
        hbuf[tt] = attn

    for i in range(5):
        _attn_step(5 * j + i)

    @pl.when(j == 1)
    def _():
        t1 = hbuf.shape[0]
        t2 = t1 - KT + 1
        hf = hbuf[...].reshape(t1 * N, cs)
        y2 = _dot(hf[:t2 * N], w2_ref[0])
        y2 = y2 + _dot(hf[N:(t2 + 1) * N], w2_ref[1])
        y2 = y2 + _dot(hf[2 * N:(t2 + 2) * N], w2_ref[2])
        co = y2.shape[-1] // 2
        g = (y2[:, :co] * jax.nn.sigmoid(y2[:, co:])).reshape(t2, N, co)
        mu = jnp.mean(g, axis=(1, 2), keepdims=True)
        var = jnp.mean((g - mu) * (g - mu), axis=(1, 2), keepdims=True)
        out_ref[0] = ((g - mu) / jnp.sqrt(var + 1e-6)) * g_ref[0, 0] \
            + bta_ref[0, 0]


def kernel(x, supports, atten_supports, W1, W_transform, W_left, W_right,
           W2, gamma, beta):
    B, T, n, cin = x.shape
    ch2 = W1.shape[-1]
    ch = ch2 // 2
    cs = W_transform.shape[-1]
    cout2 = W2.shape[-1]
    T1 = T - KT + 1
    T2 = T1 - KT + 1

    out = pl.pallas_call(
        _fused_kernel,
        grid=(B, 2),
        in_specs=[
            pl.BlockSpec((1, T, n, cin), lambda b, t: (b, 0, 0, 0)),
            pl.BlockSpec((K, n, n), lambda b, t: (0, 0, 0)),
            pl.BlockSpec((R, n, n), lambda b, t: (0, 0, 0)),
            pl.BlockSpec((KT, cin, ch2), lambda b, t: (0, 0, 0)),
            pl.BlockSpec((K, ch, cs), lambda b, t: (0, 0, 0)),
            pl.BlockSpec((K, R + 1, cs), lambda b, t: (0, 0, 0)),
            pl.BlockSpec((K, R + 1, cs), lambda b, t: (0, 0, 0)),
            pl.BlockSpec((KT, cs, cout2), lambda b, t: (0, 0, 0)),
            pl.BlockSpec((1, 1, n, cout2 // 2), lambda b, t: (0, 0, 0, 0)),
            pl.BlockSpec((1, 1, n, cout2 // 2), lambda b, t: (0, 0, 0, 0)),
        ],
        out_specs=pl.BlockSpec(
            (1, T2, n, cout2 // 2),
            lambda b, j: (b, 0, 0, 0)),
        out_shape=jax.ShapeDtypeStruct((B, T2, n, cout2 // 2), jnp.float32),
        scratch_shapes=[
            pltpu.VMEM((7 + K, n, n), jnp.bfloat16),
            pltpu.VMEM((T1, n, ch + 1), jnp.float32),
            pltpu.VMEM((T1, n, cs), jnp.float32),
            pltpu.VMEM((ch + 1, K * (cs + 1)), jnp.float32),
            pltpu.VMEM((K * (R + 1), K * (cs + 1)), jnp.float32),
            pltpu.VMEM((K * (R + 1), K * (cs + 1)), jnp.float32),
        ],
        compiler_params=pltpu.CompilerParams(
            dimension_semantics=("arbitrary", "arbitrary")),
    )(x, supports, atten_supports, W1, W_transform, W_left, W_right, W2,
      gamma, beta)
    return out
